# baseline (speedup 1.0000x reference)
import functools

import jax
import jax.numpy as jnp
from jax import lax
from jax.experimental import pallas as pl
from jax.experimental.pallas import tpu as pltpu
from jax.experimental.pallas import tpu_sc as plsc

_N = 10000
_E = 320000
_G = 100
_C = 32
_NTILES = 16
_NCORES = 2

_W = 384
_RANGE_PAD = 10368
_CH = 384
_NSUB = 3
_EPT = 2 * _RANGE_PAD
_NCH = _EPT // _CH
_S = 1536
_S4 = _S // 4
_ZSLOT = _S - 2
_PSLOT = _S - 1
_TROWS = (2 * _N) // 4


def _ranges(total):
    half = total // 2
    nw = -(-half // _W)
    big = nw - 16 * (nw // 16)
    out = []
    for sc in (0, 1):
        pos = sc * half
        for t in range(16):
            wc = nw // 16 + (1 if t < big else 0)
            end = min(pos + wc * _W, (sc + 1) * half)
            out.append((pos, end))
            pos = end
    return out

_RANGES = _ranges(_E)

_KCACHE = {}


def _make_spmm():
    mesh = plsc.VectorSubcoreMesh(
        core_axis_name="c", subcore_axis_name="s",
        num_cores=_NCORES, num_subcores=_NTILES)

    @functools.partial(
        pl.kernel,
        out_type=jax.ShapeDtypeStruct((32, _S4, 128), jnp.float32),
        mesh=mesh,
        scratch_types=[
            pltpu.VMEM((_NSUB, 128), jnp.int32),
            pltpu.VMEM((_NSUB, 128, 128), jnp.float32),
            pltpu.VMEM((_CH,), jnp.int32),
            pltpu.VMEM((_S4, 128), jnp.float32),
            pltpu.SemaphoreType.DMA,
        ],
        compiler_params=pltpu.CompilerParams(needs_layout_passes=False),
        name="sc_sorted_spmm",
    )
    def spmm(m_hbm, srcr, slotr, zeros_hbm, out_hbm,
             idx_v, rows_v, slots_v, outbuf_v, sem):
        c = lax.axis_index("c")
        s = lax.axis_index("s")
        wid = c * _NTILES + s
        pltpu.sync_copy(zeros_hbm, outbuf_v)
        iota = lax.iota(jnp.int32, 16)

        def chunk_body(ci, carry):
            pltpu.sync_copy(srcr.at[wid, ci], idx_v)
            pltpu.sync_copy(slotr.at[wid, ci], slots_v)
            descs = [pltpu.async_copy(m_hbm.at[idx_v.at[j]], rows_v.at[j], sem)
                     for j in range(_NSUB)]
            for d in descs:
                d.wait()

            def edge_body(e, carry2):
                sub = e // 128
                r = e - sub * 128
                ev = jnp.full((16,), e, jnp.int32)
                enc = plsc.load_gather(slots_v, [ev])
                q32 = (enc >> 12) * 32
                slot = enc & 0xFFF
                sv = jnp.full((16,), sub, jnp.int32)
                rv = jnp.full((16,), r, jnp.int32)
                h0 = plsc.load_gather(rows_v, [sv, rv, q32 + iota])
                h1 = plsc.load_gather(rows_v, [sv, rv, q32 + iota + 16])
                orow = slot >> 2
                ocol = (slot & 3) * 32 + iota
                plsc.addupdate_scatter(outbuf_v, [orow, ocol], h0)
                plsc.addupdate_scatter(outbuf_v, [orow, ocol + 16], h1)
                return carry2

            lax.fori_loop(0, _CH, edge_body, 0)
            return carry

        lax.fori_loop(0, _NCH, chunk_body, 0)
        pltpu.sync_copy(outbuf_v, out_hbm.at[wid])

    return spmm


def _get_spmm():
    if "spmm" not in _KCACHE:
        _KCACHE["spmm"] = _make_spmm()
    return _KCACHE["spmm"]


def _leaky(t):
    return jnp.where(t >= 0, t, 0.2 * t)


def _mm_body(h_ref, w_ref, o_ref):
    o_ref[0] = jnp.dot(h_ref[0], w_ref[0], preferred_element_type=jnp.float32)


_mm = pl.pallas_call(
    _mm_body,
    grid=(2,),
    in_specs=[pl.BlockSpec((1, _N, _C), lambda i: (i, 0, 0)),
              pl.BlockSpec((1, _C, _C), lambda i: (i, 0, 0))],
    out_specs=pl.BlockSpec((1, _N, _C), lambda i: (i, 0, 0)),
    out_shape=jax.ShapeDtypeStruct((2, _N, _C), jnp.float32),
)


def _gru_body(h_ref, agg_ref, wih_ref, whh_ref, bih_ref, bhh_ref, o_ref):
    h = h_ref[0]
    agg = agg_ref[0]
    gi = lax.dot_general(agg, wih_ref[0], (((1,), (1,)), ((), ())),
                         preferred_element_type=jnp.float32) + bih_ref[0]
    gh = lax.dot_general(h, whh_ref[0], (((1,), (1,)), ((), ())),
                         preferred_element_type=jnp.float32) + bhh_ref[0]
    r = jax.nn.sigmoid(gi[:, :_C] + gh[:, :_C])
    z = jax.nn.sigmoid(gi[:, _C:2 * _C] + gh[:, _C:2 * _C])
    cc = jnp.tanh(gi[:, 2 * _C:] + r * gh[:, 2 * _C:])
    o_ref[0] = (1.0 - z) * cc + z * h


_gru = pl.pallas_call(
    _gru_body,
    grid=(2,),
    in_specs=[pl.BlockSpec((1, _N, _C), lambda i: (i, 0, 0)),
              pl.BlockSpec((1, _N, _C), lambda i: (i, 0, 0)),
              pl.BlockSpec((1, 3 * _C, _C), lambda i: (i, 0, 0)),
              pl.BlockSpec((1, 3 * _C, _C), lambda i: (i, 0, 0)),
              pl.BlockSpec((1, 1, 3 * _C), lambda i: (i, 0, 0)),
              pl.BlockSpec((1, 1, 3 * _C), lambda i: (i, 0, 0))],
    out_specs=pl.BlockSpec((1, _N, _C), lambda i: (i, 0, 0)),
    out_shape=jax.ShapeDtypeStruct((2, _N, _C), jnp.float32),
)


def _leaky_body(h_ref, o_ref):
    o_ref[...] = _leaky(h_ref[...])


_leaky_tc = pl.pallas_call(
    _leaky_body,
    out_shape=jax.ShapeDtypeStruct((2, _N, _C), jnp.float32),
)


def _pool_body(oh_ref, h_ref, o_ref):
    o_ref[0] = jnp.dot(oh_ref[...], h_ref[0],
                       preferred_element_type=jnp.float32)


_pool = pl.pallas_call(
    _pool_body,
    grid=(2,),
    in_specs=[pl.BlockSpec((128, _N), lambda i: (0, 0)),
              pl.BlockSpec((1, _N, _C), lambda i: (i, 0, 0))],
    out_specs=pl.BlockSpec((1, 128, _C), lambda i: (i, 0, 0)),
    out_shape=jax.ShapeDtypeStruct((2, 128, _C), jnp.float32),
)


def _heads_body(hf_ref, pool_ref, inv_ref,
                tf1w, tf1b, tf2w, tf2b,
                b1w, b1b, b2w, b2b,
                v1w, v1b, v2w, v2b,
                nm1w, nm1b, nm2w, nm2b,
                nf1w, nf1b, nf2w, nf2b,
                t_ref, nm_ref, nf_ref, b_ref, v_ref):
    def mlp2(xx, w1, b1, w2, b2):
        y = _leaky(lax.dot_general(xx, w1[...], (((1,), (1,)), ((), ())),
                                   preferred_element_type=jnp.float32) + b1[...])
        return _leaky(lax.dot_general(y, w2[...], (((1,), (1,)), ((), ())),
                                      preferred_element_type=jnp.float32) + b2[...])

    pool_p = pool_ref[0] * inv_ref[...]
    pool_v = pool_ref[1] * inv_ref[...]
    t_ref[...] = mlp2(pool_p, tf1w, tf1b, tf2w, tf2b)
    b_ref[...] = mlp2(pool_p, b1w, b1b, b2w, b2b)
    v_ref[...] = mlp2(pool_v, v1w, v1b, v2w, v2b)
    nm_ref[...] = mlp2(hf_ref[0], nm1w, nm1b, nm2w, nm2b)
    nf_ref[...] = mlp2(hf_ref[0], nf1w, nf1b, nf2w, nf2b)


_heads = pl.pallas_call(
    _heads_body,
    out_shape=[
        jax.ShapeDtypeStruct((128, 128), jnp.float32),
        jax.ShapeDtypeStruct((_N, 128), jnp.float32),
        jax.ShapeDtypeStruct((_N, 128), jnp.float32),
        jax.ShapeDtypeStruct((128, 128), jnp.float32),
        jax.ShapeDtypeStruct((128, 128), jnp.float32),
    ],
)


def _pad_ggc(p):
    cc = p["w_hh"].shape[1]
    ll = p["W"].shape[0]
    wp = jnp.zeros((ll, _C, _C), jnp.float32).at[:, :cc, :cc].set(p["W"])
    wih = jnp.zeros((3 * _C, _C), jnp.float32)
    whh = jnp.zeros((3 * _C, _C), jnp.float32)
    bih = jnp.zeros((3 * _C,), jnp.float32)
    bhh = jnp.zeros((3 * _C,), jnp.float32)
    for g in range(3):
        wih = wih.at[g * _C:g * _C + cc, :cc].set(p["w_ih"][g * cc:(g + 1) * cc])
        whh = whh.at[g * _C:g * _C + cc, :cc].set(p["w_hh"][g * cc:(g + 1) * cc])
        bih = bih.at[g * _C:g * _C + cc].set(p["b_ih"][g * cc:(g + 1) * cc])
        bhh = bhh.at[g * _C:g * _C + cc].set(p["b_hh"][g * cc:(g + 1) * cc])
    return wp, wih, whh, bih, bhh


def _stack_stage(pa, pb):
    a = _pad_ggc(pa)
    b = _pad_ggc(pb)
    return (jnp.stack([a[0], b[0]], axis=1),
            jnp.stack([a[1], b[1]]),
            jnp.stack([a[2], b[2]]),
            jnp.stack([a[3], b[3]])[:, None, :],
            jnp.stack([a[4], b[4]])[:, None, :])


def _edge_setup(src, dst):
    perm = jnp.argsort(dst, stable=True)
    sd = dst[perm]
    sp = src[perm]

    nxt = jnp.concatenate([sd[1:], jnp.full((1,), -1, jnp.int32)])
    rend = jnp.zeros((_E,), jnp.bool_)
    for (a, b) in _RANGES:
        rend = rend.at[b - 1].set(True)
    flag = jnp.logical_or(sd != nxt, rend).astype(jnp.int32)
    g = jnp.cumsum(flag)
    pslot_full = jnp.zeros((_E,), jnp.int32)
    off1 = jnp.zeros((32,), jnp.int32)

    srcp = jnp.zeros((32, _RANGE_PAD), jnp.int32)
    enc0 = jnp.full((32, _RANGE_PAD), _PSLOT, jnp.int32)
    enc1 = jnp.full((32, _RANGE_PAD), _PSLOT, jnp.int32)
    for r, (a, b) in enumerate(_RANGES):
        n = b - a
        base = g[a - 1] if a > 0 else jnp.int32(0)
        nflush = g[b - 1] - base
        ps = jnp.minimum(g[a:b] - flag[a:b] - base, _S - 3)
        pslot_full = pslot_full.at[a:b].set(ps)
        off1 = off1.at[r].set(nflush)
        srcp = srcp.at[r, :n].set(sp[a:b])
        q0 = jnp.remainder(sp[a:b], 4)
        enc0 = enc0.at[r, :n].set(ps | (q0 << 12))
        ps1 = jnp.minimum(ps + nflush, _S - 3)
        q1 = jnp.remainder(sp[a:b] + _N, 4)
        enc1 = enc1.at[r, :n].set(ps1 | (q1 << 12))

    psrc0 = srcp >> 2
    psrc1 = (srcp + _N) >> 2
    srcr = jnp.concatenate([psrc0, psrc1], axis=1).reshape(
        32, _NCH, _NSUB, 128)
    slotr = jnp.concatenate([enc0, enc1], axis=1).reshape(32, _NCH, _CH)

    seg_start = jnp.searchsorted(sd, jnp.arange(_N, dtype=jnp.int32)
                                 ).astype(jnp.int32)
    seg_end = jnp.searchsorted(sd, jnp.arange(_N, dtype=jnp.int32),
                               side="right").astype(jnp.int32)
    has = seg_end > seg_start
    re_pos = jnp.asarray([b - 1 for (a, b) in _RANGES], jnp.int32)
    rs_pos = jnp.asarray([a for (a, b) in _RANGES], jnp.int32)
    j = jnp.searchsorted(re_pos, seg_start).astype(jnp.int32)
    cand = re_pos[jnp.minimum(j, 31)]
    e2 = seg_end - 1
    e1 = jnp.where(cand < e2, cand, e2)
    e1c = jnp.maximum(e1, 0)
    e2c = jnp.maximum(e2, 0)
    t1 = (jnp.searchsorted(rs_pos, e1c, side="right") - 1).astype(jnp.int32)
    t2 = (jnp.searchsorted(rs_pos, e2c, side="right") - 1).astype(jnp.int32)
    s1 = pslot_full[e1c]
    s2 = pslot_full[e2c]
    zs = jnp.int32(_ZSLOT)
    maps = []
    for tw in range(2):
        a1 = s1 + tw * off1[t1]
        a2 = s2 + tw * off1[t2]
        m1 = jnp.where(has, t1 * _S + jnp.minimum(a1, _S - 3), zs)
        m2 = jnp.where(has & (e1 != e2),
                       t2 * _S + jnp.minimum(a2, _S - 3), zs)
        maps.append((m1, m2))
    return srcr, slotr, maps


def kernel(x, edge_index, batch, params):
    src = edge_index[0].astype(jnp.int32)
    dst = edge_index[1].astype(jnp.int32)
    batch = batch.astype(jnp.int32)

    srcr, slotr, maps = _edge_setup(src, dst)
    map1 = jnp.stack([maps[0][0], maps[1][0]])
    map2 = jnp.stack([maps[0][1], maps[1][1]])
    zeros_buf = jnp.zeros((_S4, 128), jnp.float32)

    ends = jnp.searchsorted(batch, jnp.arange(_G, dtype=jnp.int32),
                            side="right").astype(jnp.int32)
    starts = jnp.concatenate([jnp.zeros((1,), jnp.int32), ends[:-1]])
    counts = ends - starts
    pos_from_end = jnp.take(ends, batch) - jnp.arange(_N, dtype=jnp.int32)
    keep_idx = jnp.nonzero(pos_from_end > 10, size=_N - 10 * _G)[0].astype(jnp.int32)
    inv_counts = 1.0 / jnp.maximum(counts, 1).astype(jnp.float32)
    inv_pad = jnp.ones((128, 1), jnp.float32).at[:_G, 0].set(inv_counts)
    onehot = (batch[None, :] == jnp.arange(128, dtype=jnp.int32)[:, None]
              ).astype(jnp.float32)

    stages = [
        _stack_stage(params["p_conv1"], params["v_conv1"]),
        _stack_stage(params["p_conv2"], params["v_conv2"]),
        _stack_stage(params["p_conv3"], params["v_conv3"]),
    ]

    h = jnp.pad(x, ((0, 0), (0, _C - x.shape[1])))
    h = jnp.broadcast_to(h[None], (2, _N, _C)).astype(jnp.float32)

    for wst, wih, whh, bih, bhh in stages:
        def body(hc, wl, wih=wih, whh=whh, bih=bih, bhh=bhh):
            m = _mm(hc, wl)
            part = _get_spmm()(m.reshape(_TROWS, 128), srcr, slotr, zeros_buf)
            pf = part.reshape(32, _S4, 4, 32).reshape(32 * _S, 32)
            agg = pf[map1] + pf[map2]
            hn = _gru(hc, agg, wih, whh, bih, bhh)
            return hn, None
        h, _ = lax.scan(body, h, wst)
        h = _leaky_tc(h)

    pool = _pool(onehot, h)

    pp = params
    ws = []
    for nm1, nm2 in (("t_fcn1", "t_fcn2"), ("b_fcn1", "b_fcn2"),
                     ("v_fcn1", "v_fcn2"), ("nmol_fcn1", "nmol_fcn2"),
                     ("nfull_fcn1", "nfull_fcn2")):
        w1p = jnp.zeros((128, 32), jnp.float32).at[:16, :].set(pp[nm1]["W"])
        b1p = jnp.zeros((1, 128), jnp.float32).at[0, :16].set(pp[nm1]["b"])
        wo = pp[nm2]["W"].shape[0]
        w2p = jnp.zeros((128, 128), jnp.float32).at[:wo, :16].set(pp[nm2]["W"])
        b2p = jnp.zeros((1, 128), jnp.float32).at[0, :wo].set(pp[nm2]["b"])
        ws += [w1p, b1p, w2p, b2p]

    t, nm_all, nf, b, v = _heads(h, pool, inv_pad, *ws)
    nm = jnp.take(nm_all[:, :1], keep_idx, axis=0)
    return (t[:_G, :2], nm, nf[:, :1], b[:_G, :3], v[:_G, :1])


# speedup vs baseline: 1.0831x; 1.0831x over previous
import functools

import jax
import jax.numpy as jnp
from jax import lax
from jax.experimental import pallas as pl
from jax.experimental.pallas import tpu as pltpu
from jax.experimental.pallas import tpu_sc as plsc

_N = 10000
_E = 320000
_G = 100
_C = 32
_NTILES = 16
_NCORES = 2

_W = 384
_RANGE_PAD = 10368
_CH = 384
_NSUB = 3
_EPT = 2 * _RANGE_PAD
_NCH = _EPT // _CH
_S = 1536
_S4 = _S // 4
_ZSLOT = _S - 2
_PSLOT = _S - 1
_TROWS = (2 * _N) // 4


def _ranges(total):
    half = total // 2
    nw = -(-half // _W)
    big = nw - 16 * (nw // 16)
    out = []
    for sc in (0, 1):
        pos = sc * half
        for t in range(16):
            wc = nw // 16 + (1 if t < big else 0)
            end = min(pos + wc * _W, (sc + 1) * half)
            out.append((pos, end))
            pos = end
    return out

_RANGES = _ranges(_E)

_KCACHE = {}


def _make_spmm():
    mesh = plsc.VectorSubcoreMesh(
        core_axis_name="c", subcore_axis_name="s",
        num_cores=_NCORES, num_subcores=_NTILES)

    @functools.partial(
        pl.kernel,
        out_type=jax.ShapeDtypeStruct((32, _S4, 128), jnp.float32),
        mesh=mesh,
        scratch_types=[
            pltpu.VMEM((_NSUB, 128), jnp.int32),
            pltpu.VMEM((_NSUB, 128, 128), jnp.float32),
            pltpu.VMEM((_CH,), jnp.int32),
            pltpu.VMEM((_S4, 128), jnp.float32),
            pltpu.SemaphoreType.DMA,
        ],
        compiler_params=pltpu.CompilerParams(needs_layout_passes=False),
        name="sc_sorted_spmm",
    )
    def spmm(m_hbm, srcr, slotr, zeros_hbm, out_hbm,
             idx_v, rows_v, slots_v, outbuf_v, sem):
        c = lax.axis_index("c")
        s = lax.axis_index("s")
        wid = c * _NTILES + s
        pltpu.sync_copy(zeros_hbm, outbuf_v)
        iota = lax.iota(jnp.int32, 16)

        def chunk_body(ci, carry):
            pltpu.sync_copy(srcr.at[wid, ci], idx_v)
            pltpu.sync_copy(slotr.at[wid, ci], slots_v)
            descs = [pltpu.async_copy(m_hbm.at[idx_v.at[j]], rows_v.at[j], sem)
                     for j in range(_NSUB)]
            for d in descs:
                d.wait()

            def edge_body(eg, carry2):
                e0 = eg * 8
                gathered = []
                for u in range(8):
                    e = e0 + u
                    sub = e // 128
                    r = e - sub * 128
                    ev = jnp.full((16,), e, jnp.int32)
                    enc = plsc.load_gather(slots_v, [ev])
                    q32 = (enc >> 12) * 32
                    slot = enc & 0xFFF
                    sv = jnp.full((16,), sub, jnp.int32)
                    rv = jnp.full((16,), r, jnp.int32)
                    h0 = plsc.load_gather(rows_v, [sv, rv, q32 + iota])
                    h1 = plsc.load_gather(rows_v, [sv, rv, q32 + iota + 16])
                    gathered.append((slot, h0, h1))
                for slot, h0, h1 in gathered:
                    orow = slot >> 2
                    ocol = (slot & 3) * 32 + iota
                    plsc.addupdate_scatter(outbuf_v, [orow, ocol], h0)
                    plsc.addupdate_scatter(outbuf_v, [orow, ocol + 16], h1)
                return carry2

            lax.fori_loop(0, _CH // 8, edge_body, 0)
            return carry

        lax.fori_loop(0, _NCH, chunk_body, 0)
        pltpu.sync_copy(outbuf_v, out_hbm.at[wid])

    return spmm


def _get_spmm():
    if "spmm" not in _KCACHE:
        _KCACHE["spmm"] = _make_spmm()
    return _KCACHE["spmm"]


def _leaky(t):
    return jnp.where(t >= 0, t, 0.2 * t)


def _mm_body(h_ref, w_ref, o_ref):
    o_ref[0] = jnp.dot(h_ref[0], w_ref[0], preferred_element_type=jnp.float32)


_mm = pl.pallas_call(
    _mm_body,
    grid=(2,),
    in_specs=[pl.BlockSpec((1, _N, _C), lambda i: (i, 0, 0)),
              pl.BlockSpec((1, _C, _C), lambda i: (i, 0, 0))],
    out_specs=pl.BlockSpec((1, _N, _C), lambda i: (i, 0, 0)),
    out_shape=jax.ShapeDtypeStruct((2, _N, _C), jnp.float32),
)


def _gru_body(h_ref, agg_ref, wih_ref, whh_ref, bih_ref, bhh_ref, o_ref):
    h = h_ref[0]
    agg = agg_ref[0]
    gi = lax.dot_general(agg, wih_ref[0], (((1,), (1,)), ((), ())),
                         preferred_element_type=jnp.float32) + bih_ref[0]
    gh = lax.dot_general(h, whh_ref[0], (((1,), (1,)), ((), ())),
                         preferred_element_type=jnp.float32) + bhh_ref[0]
    r = jax.nn.sigmoid(gi[:, :_C] + gh[:, :_C])
    z = jax.nn.sigmoid(gi[:, _C:2 * _C] + gh[:, _C:2 * _C])
    cc = jnp.tanh(gi[:, 2 * _C:] + r * gh[:, 2 * _C:])
    o_ref[0] = (1.0 - z) * cc + z * h


_gru = pl.pallas_call(
    _gru_body,
    grid=(2,),
    in_specs=[pl.BlockSpec((1, _N, _C), lambda i: (i, 0, 0)),
              pl.BlockSpec((1, _N, _C), lambda i: (i, 0, 0)),
              pl.BlockSpec((1, 3 * _C, _C), lambda i: (i, 0, 0)),
              pl.BlockSpec((1, 3 * _C, _C), lambda i: (i, 0, 0)),
              pl.BlockSpec((1, 1, 3 * _C), lambda i: (i, 0, 0)),
              pl.BlockSpec((1, 1, 3 * _C), lambda i: (i, 0, 0))],
    out_specs=pl.BlockSpec((1, _N, _C), lambda i: (i, 0, 0)),
    out_shape=jax.ShapeDtypeStruct((2, _N, _C), jnp.float32),
)


def _leaky_body(h_ref, o_ref):
    o_ref[...] = _leaky(h_ref[...])


_leaky_tc = pl.pallas_call(
    _leaky_body,
    out_shape=jax.ShapeDtypeStruct((2, _N, _C), jnp.float32),
)


def _pool_body(oh_ref, h_ref, o_ref):
    o_ref[0] = jnp.dot(oh_ref[...], h_ref[0],
                       preferred_element_type=jnp.float32)


_pool = pl.pallas_call(
    _pool_body,
    grid=(2,),
    in_specs=[pl.BlockSpec((128, _N), lambda i: (0, 0)),
              pl.BlockSpec((1, _N, _C), lambda i: (i, 0, 0))],
    out_specs=pl.BlockSpec((1, 128, _C), lambda i: (i, 0, 0)),
    out_shape=jax.ShapeDtypeStruct((2, 128, _C), jnp.float32),
)


def _heads_body(hf_ref, pool_ref, inv_ref,
                tf1w, tf1b, tf2w, tf2b,
                b1w, b1b, b2w, b2b,
                v1w, v1b, v2w, v2b,
                nm1w, nm1b, nm2w, nm2b,
                nf1w, nf1b, nf2w, nf2b,
                t_ref, nm_ref, nf_ref, b_ref, v_ref):
    def mlp2(xx, w1, b1, w2, b2):
        y = _leaky(lax.dot_general(xx, w1[...], (((1,), (1,)), ((), ())),
                                   preferred_element_type=jnp.float32) + b1[...])
        return _leaky(lax.dot_general(y, w2[...], (((1,), (1,)), ((), ())),
                                      preferred_element_type=jnp.float32) + b2[...])

    pool_p = pool_ref[0] * inv_ref[...]
    pool_v = pool_ref[1] * inv_ref[...]
    t_ref[...] = mlp2(pool_p, tf1w, tf1b, tf2w, tf2b)
    b_ref[...] = mlp2(pool_p, b1w, b1b, b2w, b2b)
    v_ref[...] = mlp2(pool_v, v1w, v1b, v2w, v2b)
    nm_ref[...] = mlp2(hf_ref[0], nm1w, nm1b, nm2w, nm2b)
    nf_ref[...] = mlp2(hf_ref[0], nf1w, nf1b, nf2w, nf2b)


_heads = pl.pallas_call(
    _heads_body,
    out_shape=[
        jax.ShapeDtypeStruct((128, 128), jnp.float32),
        jax.ShapeDtypeStruct((_N, 128), jnp.float32),
        jax.ShapeDtypeStruct((_N, 128), jnp.float32),
        jax.ShapeDtypeStruct((128, 128), jnp.float32),
        jax.ShapeDtypeStruct((128, 128), jnp.float32),
    ],
)


def _pad_ggc(p):
    cc = p["w_hh"].shape[1]
    ll = p["W"].shape[0]
    wp = jnp.zeros((ll, _C, _C), jnp.float32).at[:, :cc, :cc].set(p["W"])
    wih = jnp.zeros((3 * _C, _C), jnp.float32)
    whh = jnp.zeros((3 * _C, _C), jnp.float32)
    bih = jnp.zeros((3 * _C,), jnp.float32)
    bhh = jnp.zeros((3 * _C,), jnp.float32)
    for g in range(3):
        wih = wih.at[g * _C:g * _C + cc, :cc].set(p["w_ih"][g * cc:(g + 1) * cc])
        whh = whh.at[g * _C:g * _C + cc, :cc].set(p["w_hh"][g * cc:(g + 1) * cc])
        bih = bih.at[g * _C:g * _C + cc].set(p["b_ih"][g * cc:(g + 1) * cc])
        bhh = bhh.at[g * _C:g * _C + cc].set(p["b_hh"][g * cc:(g + 1) * cc])
    return wp, wih, whh, bih, bhh


def _stack_stage(pa, pb):
    a = _pad_ggc(pa)
    b = _pad_ggc(pb)
    return (jnp.stack([a[0], b[0]], axis=1),
            jnp.stack([a[1], b[1]]),
            jnp.stack([a[2], b[2]]),
            jnp.stack([a[3], b[3]])[:, None, :],
            jnp.stack([a[4], b[4]])[:, None, :])


def _edge_setup(src, dst):
    perm = jnp.argsort(dst, stable=True)
    sd = dst[perm]
    sp = src[perm]

    nxt = jnp.concatenate([sd[1:], jnp.full((1,), -1, jnp.int32)])
    rend = jnp.zeros((_E,), jnp.bool_)
    for (a, b) in _RANGES:
        rend = rend.at[b - 1].set(True)
    flag = jnp.logical_or(sd != nxt, rend).astype(jnp.int32)
    g = jnp.cumsum(flag)
    pslot_full = jnp.zeros((_E,), jnp.int32)
    off1 = jnp.zeros((32,), jnp.int32)

    srcp = jnp.zeros((32, _RANGE_PAD), jnp.int32)
    enc0 = jnp.full((32, _RANGE_PAD), _PSLOT, jnp.int32)
    enc1 = jnp.full((32, _RANGE_PAD), _PSLOT, jnp.int32)
    for r, (a, b) in enumerate(_RANGES):
        n = b - a
        base = g[a - 1] if a > 0 else jnp.int32(0)
        nflush = g[b - 1] - base
        ps = jnp.minimum(g[a:b] - flag[a:b] - base, _S - 3)
        pslot_full = pslot_full.at[a:b].set(ps)
        off1 = off1.at[r].set(nflush)
        srcp = srcp.at[r, :n].set(sp[a:b])
        q0 = jnp.remainder(sp[a:b], 4)
        enc0 = enc0.at[r, :n].set(ps | (q0 << 12))
        ps1 = jnp.minimum(ps + nflush, _S - 3)
        q1 = jnp.remainder(sp[a:b] + _N, 4)
        enc1 = enc1.at[r, :n].set(ps1 | (q1 << 12))

    psrc0 = srcp >> 2
    psrc1 = (srcp + _N) >> 2
    srcr = jnp.concatenate([psrc0, psrc1], axis=1).reshape(
        32, _NCH, _NSUB, 128)
    slotr = jnp.concatenate([enc0, enc1], axis=1).reshape(32, _NCH, _CH)

    seg_start = jnp.searchsorted(sd, jnp.arange(_N, dtype=jnp.int32)
                                 ).astype(jnp.int32)
    seg_end = jnp.searchsorted(sd, jnp.arange(_N, dtype=jnp.int32),
                               side="right").astype(jnp.int32)
    has = seg_end > seg_start
    re_pos = jnp.asarray([b - 1 for (a, b) in _RANGES], jnp.int32)
    rs_pos = jnp.asarray([a for (a, b) in _RANGES], jnp.int32)
    j = jnp.searchsorted(re_pos, seg_start).astype(jnp.int32)
    cand = re_pos[jnp.minimum(j, 31)]
    e2 = seg_end - 1
    e1 = jnp.where(cand < e2, cand, e2)
    e1c = jnp.maximum(e1, 0)
    e2c = jnp.maximum(e2, 0)
    t1 = (jnp.searchsorted(rs_pos, e1c, side="right") - 1).astype(jnp.int32)
    t2 = (jnp.searchsorted(rs_pos, e2c, side="right") - 1).astype(jnp.int32)
    s1 = pslot_full[e1c]
    s2 = pslot_full[e2c]
    zs = jnp.int32(_ZSLOT)
    maps = []
    for tw in range(2):
        a1 = s1 + tw * off1[t1]
        a2 = s2 + tw * off1[t2]
        m1 = jnp.where(has, t1 * _S + jnp.minimum(a1, _S - 3), zs)
        m2 = jnp.where(has & (e1 != e2),
                       t2 * _S + jnp.minimum(a2, _S - 3), zs)
        maps.append((m1, m2))
    return srcr, slotr, maps


def kernel(x, edge_index, batch, params):
    src = edge_index[0].astype(jnp.int32)
    dst = edge_index[1].astype(jnp.int32)
    batch = batch.astype(jnp.int32)

    srcr, slotr, maps = _edge_setup(src, dst)
    map1 = jnp.stack([maps[0][0], maps[1][0]])
    map2 = jnp.stack([maps[0][1], maps[1][1]])
    zeros_buf = jnp.zeros((_S4, 128), jnp.float32)

    ends = jnp.searchsorted(batch, jnp.arange(_G, dtype=jnp.int32),
                            side="right").astype(jnp.int32)
    starts = jnp.concatenate([jnp.zeros((1,), jnp.int32), ends[:-1]])
    counts = ends - starts
    pos_from_end = jnp.take(ends, batch) - jnp.arange(_N, dtype=jnp.int32)
    keep_idx = jnp.nonzero(pos_from_end > 10, size=_N - 10 * _G)[0].astype(jnp.int32)
    inv_counts = 1.0 / jnp.maximum(counts, 1).astype(jnp.float32)
    inv_pad = jnp.ones((128, 1), jnp.float32).at[:_G, 0].set(inv_counts)
    onehot = (batch[None, :] == jnp.arange(128, dtype=jnp.int32)[:, None]
              ).astype(jnp.float32)

    stages = [
        _stack_stage(params["p_conv1"], params["v_conv1"]),
        _stack_stage(params["p_conv2"], params["v_conv2"]),
        _stack_stage(params["p_conv3"], params["v_conv3"]),
    ]

    h = jnp.pad(x, ((0, 0), (0, _C - x.shape[1])))
    h = jnp.broadcast_to(h[None], (2, _N, _C)).astype(jnp.float32)

    for wst, wih, whh, bih, bhh in stages:
        def body(hc, wl, wih=wih, whh=whh, bih=bih, bhh=bhh):
            m = _mm(hc, wl)
            part = _get_spmm()(m.reshape(_TROWS, 128), srcr, slotr, zeros_buf)
            pf = part.reshape(32, _S4, 4, 32).reshape(32 * _S, 32)
            agg = pf[map1] + pf[map2]
            hn = _gru(hc, agg, wih, whh, bih, bhh)
            return hn, None
        h, _ = lax.scan(body, h, wst)
        h = _leaky_tc(h)

    pool = _pool(onehot, h)

    pp = params
    ws = []
    for nm1, nm2 in (("t_fcn1", "t_fcn2"), ("b_fcn1", "b_fcn2"),
                     ("v_fcn1", "v_fcn2"), ("nmol_fcn1", "nmol_fcn2"),
                     ("nfull_fcn1", "nfull_fcn2")):
        w1p = jnp.zeros((128, 32), jnp.float32).at[:16, :].set(pp[nm1]["W"])
        b1p = jnp.zeros((1, 128), jnp.float32).at[0, :16].set(pp[nm1]["b"])
        wo = pp[nm2]["W"].shape[0]
        w2p = jnp.zeros((128, 128), jnp.float32).at[:wo, :16].set(pp[nm2]["W"])
        b2p = jnp.zeros((1, 128), jnp.float32).at[0, :wo].set(pp[nm2]["b"])
        ws += [w1p, b1p, w2p, b2p]

    t, nm_all, nf, b, v = _heads(h, pool, inv_pad, *ws)
    nm = jnp.take(nm_all[:, :1], keep_idx, axis=0)
    return (t[:_G, :2], nm, nf[:, :1], b[:_G, :3], v[:_G, :1])


# speedup vs baseline: 1.0831x; 1.0000x over previous
import functools

import jax
import jax.numpy as jnp
from jax import lax
from jax.experimental import pallas as pl
from jax.experimental.pallas import tpu as pltpu
from jax.experimental.pallas import tpu_sc as plsc

_N = 10000
_E = 320000
_G = 100
_C = 32
_NTILES = 16
_NCORES = 2

_W = 384
_RANGE_PAD = 10368
_CH = 384
_NSUB = 3
_EPT = 2 * _RANGE_PAD
_NCH = _EPT // _CH
_S = 1536
_S4 = _S // 4
_ZSLOT = _S - 2
_PSLOT = _S - 1
_TROWS = (2 * _N) // 4


def _ranges(total):
    half = total // 2
    nw = -(-half // _W)
    big = nw - 16 * (nw // 16)
    out = []
    for sc in (0, 1):
        pos = sc * half
        for t in range(16):
            wc = nw // 16 + (1 if t < big else 0)
            end = min(pos + wc * _W, (sc + 1) * half)
            out.append((pos, end))
            pos = end
    return out

_RANGES = _ranges(_E)

_KCACHE = {}


def _make_spmm():
    mesh = plsc.VectorSubcoreMesh(
        core_axis_name="c", subcore_axis_name="s",
        num_cores=_NCORES, num_subcores=_NTILES)

    @functools.partial(
        pl.kernel,
        out_type=jax.ShapeDtypeStruct((32, _S4, 128), jnp.float32),
        mesh=mesh,
        scratch_types=[
            pltpu.VMEM((_NSUB, 128), jnp.int32),
            pltpu.VMEM((_NSUB, 128, 128), jnp.float32),
            pltpu.VMEM((_CH,), jnp.int32),
            pltpu.VMEM((_S4, 128), jnp.float32),
            pltpu.SemaphoreType.DMA,
        ],
        compiler_params=pltpu.CompilerParams(needs_layout_passes=False),
        name="sc_sorted_spmm",
    )
    def spmm(m_hbm, srcr, slotr, zeros_hbm, out_hbm,
             idx_v, rows_v, slots_v, outbuf_v, sem):
        c = lax.axis_index("c")
        s = lax.axis_index("s")
        wid = c * _NTILES + s
        pltpu.sync_copy(zeros_hbm, outbuf_v)
        iota = lax.iota(jnp.int32, 16)

        def chunk_body(ci, carry):
            pltpu.sync_copy(srcr.at[wid, ci], idx_v)
            pltpu.sync_copy(slotr.at[wid, ci], slots_v)
            descs = [pltpu.async_copy(m_hbm.at[idx_v.at[j]], rows_v.at[j], sem)
                     for j in range(_NSUB)]
            for d in descs:
                d.wait()

            def edge_body(eg, carry2):
                e0 = eg * 8
                gathered = []
                for u in range(8):
                    e = e0 + u
                    sub = e // 128
                    r = e - sub * 128
                    ev = jnp.full((16,), e, jnp.int32)
                    enc = plsc.load_gather(slots_v, [ev])
                    q32 = (enc >> 12) * 32
                    slot = enc & 0xFFF
                    sv = jnp.full((16,), sub, jnp.int32)
                    rv = jnp.full((16,), r, jnp.int32)
                    h0 = plsc.load_gather(rows_v, [sv, rv, q32 + iota])
                    h1 = plsc.load_gather(rows_v, [sv, rv, q32 + iota + 16])
                    gathered.append((slot, h0, h1))
                for slot, h0, h1 in gathered:
                    orow = slot >> 2
                    ocol = (slot & 3) * 32 + iota
                    plsc.addupdate_scatter(outbuf_v, [orow, ocol], h0)
                    plsc.addupdate_scatter(outbuf_v, [orow, ocol + 16], h1)
                return carry2

            lax.fori_loop(0, _CH // 8, edge_body, 0)
            return carry

        lax.fori_loop(0, _NCH, chunk_body, 0)
        pltpu.sync_copy(outbuf_v, out_hbm.at[wid])

    return spmm


def _get_spmm():
    if "spmm" not in _KCACHE:
        _KCACHE["spmm"] = _make_spmm()
    return _KCACHE["spmm"]


def _leaky(t):
    return jnp.where(t >= 0, t, 0.2 * t)


def _mm_body(h_ref, w_ref, o_ref):
    o_ref[0] = jnp.dot(h_ref[0], w_ref[0], preferred_element_type=jnp.float32)


_mm = pl.pallas_call(
    _mm_body,
    grid=(2,),
    in_specs=[pl.BlockSpec((1, _N, _C), lambda i: (i, 0, 0)),
              pl.BlockSpec((1, _C, _C), lambda i: (i, 0, 0))],
    out_specs=pl.BlockSpec((1, _N, _C), lambda i: (i, 0, 0)),
    out_shape=jax.ShapeDtypeStruct((2, _N, _C), jnp.float32),
)


def _gru_body(h_ref, agg_ref, wih_ref, whh_ref, bih_ref, bhh_ref, o_ref):
    h = h_ref[0]
    agg = agg_ref[0]
    gi = lax.dot_general(agg, wih_ref[0], (((1,), (1,)), ((), ())),
                         preferred_element_type=jnp.float32) + bih_ref[0]
    gh = lax.dot_general(h, whh_ref[0], (((1,), (1,)), ((), ())),
                         preferred_element_type=jnp.float32) + bhh_ref[0]
    r = jax.nn.sigmoid(gi[:, :_C] + gh[:, :_C])
    z = jax.nn.sigmoid(gi[:, _C:2 * _C] + gh[:, _C:2 * _C])
    cc = jnp.tanh(gi[:, 2 * _C:] + r * gh[:, 2 * _C:])
    o_ref[0] = (1.0 - z) * cc + z * h


_gru = pl.pallas_call(
    _gru_body,
    grid=(2,),
    in_specs=[pl.BlockSpec((1, _N, _C), lambda i: (i, 0, 0)),
              pl.BlockSpec((1, _N, _C), lambda i: (i, 0, 0)),
              pl.BlockSpec((1, 3 * _C, _C), lambda i: (i, 0, 0)),
              pl.BlockSpec((1, 3 * _C, _C), lambda i: (i, 0, 0)),
              pl.BlockSpec((1, 1, 3 * _C), lambda i: (i, 0, 0)),
              pl.BlockSpec((1, 1, 3 * _C), lambda i: (i, 0, 0))],
    out_specs=pl.BlockSpec((1, _N, _C), lambda i: (i, 0, 0)),
    out_shape=jax.ShapeDtypeStruct((2, _N, _C), jnp.float32),
)


def _leaky_body(h_ref, o_ref):
    o_ref[...] = _leaky(h_ref[...])


_leaky_tc = pl.pallas_call(
    _leaky_body,
    out_shape=jax.ShapeDtypeStruct((2, _N, _C), jnp.float32),
)


def _pool_body(oh_ref, h_ref, o_ref):
    o_ref[0] = jnp.dot(oh_ref[...], h_ref[0],
                       precision=lax.Precision.HIGHEST,
                       preferred_element_type=jnp.float32)


_pool = pl.pallas_call(
    _pool_body,
    grid=(2,),
    in_specs=[pl.BlockSpec((128, _N), lambda i: (0, 0)),
              pl.BlockSpec((1, _N, _C), lambda i: (i, 0, 0))],
    out_specs=pl.BlockSpec((1, 128, _C), lambda i: (i, 0, 0)),
    out_shape=jax.ShapeDtypeStruct((2, 128, _C), jnp.float32),
)


def _heads_body(hf_ref, pool_ref, inv_ref,
                tf1w, tf1b, tf2w, tf2b,
                b1w, b1b, b2w, b2b,
                v1w, v1b, v2w, v2b,
                nm1w, nm1b, nm2w, nm2b,
                nf1w, nf1b, nf2w, nf2b,
                t_ref, nm_ref, nf_ref, b_ref, v_ref):
    def mlp2(xx, w1, b1, w2, b2):
        y = _leaky(lax.dot_general(xx, w1[...], (((1,), (1,)), ((), ())),
                                   preferred_element_type=jnp.float32) + b1[...])
        return _leaky(lax.dot_general(y, w2[...], (((1,), (1,)), ((), ())),
                                      preferred_element_type=jnp.float32) + b2[...])

    pool_p = pool_ref[0] * inv_ref[...]
    pool_v = pool_ref[1] * inv_ref[...]
    t_ref[...] = mlp2(pool_p, tf1w, tf1b, tf2w, tf2b)
    b_ref[...] = mlp2(pool_p, b1w, b1b, b2w, b2b)
    v_ref[...] = mlp2(pool_v, v1w, v1b, v2w, v2b)
    nm_ref[...] = mlp2(hf_ref[0], nm1w, nm1b, nm2w, nm2b)
    nf_ref[...] = mlp2(hf_ref[0], nf1w, nf1b, nf2w, nf2b)


_heads = pl.pallas_call(
    _heads_body,
    out_shape=[
        jax.ShapeDtypeStruct((128, 128), jnp.float32),
        jax.ShapeDtypeStruct((_N, 128), jnp.float32),
        jax.ShapeDtypeStruct((_N, 128), jnp.float32),
        jax.ShapeDtypeStruct((128, 128), jnp.float32),
        jax.ShapeDtypeStruct((128, 128), jnp.float32),
    ],
)


def _pad_ggc(p):
    cc = p["w_hh"].shape[1]
    ll = p["W"].shape[0]
    wp = jnp.zeros((ll, _C, _C), jnp.float32).at[:, :cc, :cc].set(p["W"])
    wih = jnp.zeros((3 * _C, _C), jnp.float32)
    whh = jnp.zeros((3 * _C, _C), jnp.float32)
    bih = jnp.zeros((3 * _C,), jnp.float32)
    bhh = jnp.zeros((3 * _C,), jnp.float32)
    for g in range(3):
        wih = wih.at[g * _C:g * _C + cc, :cc].set(p["w_ih"][g * cc:(g + 1) * cc])
        whh = whh.at[g * _C:g * _C + cc, :cc].set(p["w_hh"][g * cc:(g + 1) * cc])
        bih = bih.at[g * _C:g * _C + cc].set(p["b_ih"][g * cc:(g + 1) * cc])
        bhh = bhh.at[g * _C:g * _C + cc].set(p["b_hh"][g * cc:(g + 1) * cc])
    return wp, wih, whh, bih, bhh


def _stack_stage(pa, pb):
    a = _pad_ggc(pa)
    b = _pad_ggc(pb)
    return (jnp.stack([a[0], b[0]], axis=1),
            jnp.stack([a[1], b[1]]),
            jnp.stack([a[2], b[2]]),
            jnp.stack([a[3], b[3]])[:, None, :],
            jnp.stack([a[4], b[4]])[:, None, :])


def _edge_setup(src, dst):
    perm = jnp.argsort(dst, stable=True)
    sd = dst[perm]
    sp = src[perm]

    nxt = jnp.concatenate([sd[1:], jnp.full((1,), -1, jnp.int32)])
    rend = jnp.zeros((_E,), jnp.bool_)
    for (a, b) in _RANGES:
        rend = rend.at[b - 1].set(True)
    flag = jnp.logical_or(sd != nxt, rend).astype(jnp.int32)
    g = jnp.cumsum(flag)
    pslot_full = jnp.zeros((_E,), jnp.int32)
    off1 = jnp.zeros((32,), jnp.int32)

    srcp = jnp.zeros((32, _RANGE_PAD), jnp.int32)
    enc0 = jnp.full((32, _RANGE_PAD), _PSLOT, jnp.int32)
    enc1 = jnp.full((32, _RANGE_PAD), _PSLOT, jnp.int32)
    for r, (a, b) in enumerate(_RANGES):
        n = b - a
        base = g[a - 1] if a > 0 else jnp.int32(0)
        nflush = g[b - 1] - base
        ps = jnp.minimum(g[a:b] - flag[a:b] - base, _S - 3)
        pslot_full = pslot_full.at[a:b].set(ps)
        off1 = off1.at[r].set(nflush)
        srcp = srcp.at[r, :n].set(sp[a:b])
        q0 = jnp.remainder(sp[a:b], 4)
        enc0 = enc0.at[r, :n].set(ps | (q0 << 12))
        ps1 = jnp.minimum(ps + nflush, _S - 3)
        q1 = jnp.remainder(sp[a:b] + _N, 4)
        enc1 = enc1.at[r, :n].set(ps1 | (q1 << 12))

    psrc0 = srcp >> 2
    psrc1 = (srcp + _N) >> 2
    srcr = jnp.concatenate([psrc0, psrc1], axis=1).reshape(
        32, _NCH, _NSUB, 128)
    slotr = jnp.concatenate([enc0, enc1], axis=1).reshape(32, _NCH, _CH)

    seg_start = jnp.searchsorted(sd, jnp.arange(_N, dtype=jnp.int32)
                                 ).astype(jnp.int32)
    seg_end = jnp.searchsorted(sd, jnp.arange(_N, dtype=jnp.int32),
                               side="right").astype(jnp.int32)
    has = seg_end > seg_start
    re_pos = jnp.asarray([b - 1 for (a, b) in _RANGES], jnp.int32)
    rs_pos = jnp.asarray([a for (a, b) in _RANGES], jnp.int32)
    j = jnp.searchsorted(re_pos, seg_start).astype(jnp.int32)
    cand = re_pos[jnp.minimum(j, 31)]
    e2 = seg_end - 1
    e1 = jnp.where(cand < e2, cand, e2)
    e1c = jnp.maximum(e1, 0)
    e2c = jnp.maximum(e2, 0)
    t1 = (jnp.searchsorted(rs_pos, e1c, side="right") - 1).astype(jnp.int32)
    t2 = (jnp.searchsorted(rs_pos, e2c, side="right") - 1).astype(jnp.int32)
    s1 = pslot_full[e1c]
    s2 = pslot_full[e2c]
    zs = jnp.int32(_ZSLOT)
    maps = []
    for tw in range(2):
        a1 = s1 + tw * off1[t1]
        a2 = s2 + tw * off1[t2]
        m1 = jnp.where(has, t1 * _S + jnp.minimum(a1, _S - 3), zs)
        m2 = jnp.where(has & (e1 != e2),
                       t2 * _S + jnp.minimum(a2, _S - 3), zs)
        maps.append((m1, m2))
    return srcr, slotr, maps


def kernel(x, edge_index, batch, params):
    src = edge_index[0].astype(jnp.int32)
    dst = edge_index[1].astype(jnp.int32)
    batch = batch.astype(jnp.int32)

    srcr, slotr, maps = _edge_setup(src, dst)
    map1 = jnp.stack([maps[0][0], maps[1][0]])
    map2 = jnp.stack([maps[0][1], maps[1][1]])
    zeros_buf = jnp.zeros((_S4, 128), jnp.float32)

    ends = jnp.searchsorted(batch, jnp.arange(_G, dtype=jnp.int32),
                            side="right").astype(jnp.int32)
    starts = jnp.concatenate([jnp.zeros((1,), jnp.int32), ends[:-1]])
    counts = ends - starts
    pos_from_end = jnp.take(ends, batch) - jnp.arange(_N, dtype=jnp.int32)
    keep_idx = jnp.nonzero(pos_from_end > 10, size=_N - 10 * _G)[0].astype(jnp.int32)
    inv_counts = 1.0 / jnp.maximum(counts, 1).astype(jnp.float32)
    inv_pad = jnp.ones((128, 1), jnp.float32).at[:_G, 0].set(inv_counts)
    onehot = (batch[None, :] == jnp.arange(128, dtype=jnp.int32)[:, None]
              ).astype(jnp.float32)

    stages = [
        _stack_stage(params["p_conv1"], params["v_conv1"]),
        _stack_stage(params["p_conv2"], params["v_conv2"]),
        _stack_stage(params["p_conv3"], params["v_conv3"]),
    ]

    h = jnp.pad(x, ((0, 0), (0, _C - x.shape[1])))
    h = jnp.broadcast_to(h[None], (2, _N, _C)).astype(jnp.float32)

    for wst, wih, whh, bih, bhh in stages:
        def body(hc, wl, wih=wih, whh=whh, bih=bih, bhh=bhh):
            m = _mm(hc, wl)
            part = _get_spmm()(m.reshape(_TROWS, 128), srcr, slotr, zeros_buf)
            pf = part.reshape(32, _S4, 4, 32).reshape(32 * _S, 32)
            agg = pf[map1] + pf[map2]
            hn = _gru(hc, agg, wih, whh, bih, bhh)
            return hn, None
        h, _ = lax.scan(body, h, wst)
        h = _leaky_tc(h)

    pool = _pool(onehot, h)

    pp = params
    ws = []
    for nm1, nm2 in (("t_fcn1", "t_fcn2"), ("b_fcn1", "b_fcn2"),
                     ("v_fcn1", "v_fcn2"), ("nmol_fcn1", "nmol_fcn2"),
                     ("nfull_fcn1", "nfull_fcn2")):
        w1p = jnp.zeros((128, 32), jnp.float32).at[:16, :].set(pp[nm1]["W"])
        b1p = jnp.zeros((1, 128), jnp.float32).at[0, :16].set(pp[nm1]["b"])
        wo = pp[nm2]["W"].shape[0]
        w2p = jnp.zeros((128, 128), jnp.float32).at[:wo, :16].set(pp[nm2]["W"])
        b2p = jnp.zeros((1, 128), jnp.float32).at[0, :wo].set(pp[nm2]["b"])
        ws += [w1p, b1p, w2p, b2p]

    t, nm_all, nf, b, v = _heads(h, pool, inv_pad, *ws)
    nm = jnp.take(nm_all[:, :1], keep_idx, axis=0)
    return (t[:_G, :2], nm, nf[:, :1], b[:_G, :3], v[:_G, :1])


# speedup vs baseline: 1.0873x; 1.0039x over previous
import functools

import jax
import jax.numpy as jnp
from jax import lax
from jax.experimental import pallas as pl
from jax.experimental.pallas import tpu as pltpu
from jax.experimental.pallas import tpu_sc as plsc

_N = 10000
_E = 320000
_G = 100
_C = 32
_NTILES = 16
_NCORES = 2

_W = 384
_RANGE_PAD = 10368
_CH = 384
_NSUB = 3
_EPT = 2 * _RANGE_PAD
_NCH = _EPT // _CH
_S = 1536
_S4 = _S // 4
_ZSLOT = _S - 2
_PSLOT = _S - 1
_TROWS = (2 * _N) // 4


def _ranges(total):
    half = total // 2
    nw = -(-half // _W)
    big = nw - 16 * (nw // 16)
    out = []
    for sc in (0, 1):
        pos = sc * half
        for t in range(16):
            wc = nw // 16 + (1 if t < big else 0)
            end = min(pos + wc * _W, (sc + 1) * half)
            out.append((pos, end))
            pos = end
    return out

_RANGES = _ranges(_E)

_KCACHE = {}


def _make_spmm():
    mesh = plsc.VectorSubcoreMesh(
        core_axis_name="c", subcore_axis_name="s",
        num_cores=_NCORES, num_subcores=_NTILES)

    @functools.partial(
        pl.kernel,
        out_type=jax.ShapeDtypeStruct((32, _S4, 128), jnp.float32),
        mesh=mesh,
        scratch_types=[
            pltpu.VMEM((_NSUB, 128), jnp.int32),
            pltpu.VMEM((_NSUB, 128, 128), jnp.float32),
            pltpu.VMEM((_CH,), jnp.int32),
            pltpu.VMEM((_S4, 128), jnp.float32),
            pltpu.SemaphoreType.DMA,
        ],
        compiler_params=pltpu.CompilerParams(needs_layout_passes=False),
        name="sc_sorted_spmm",
    )
    def spmm(m_hbm, srcr, slotr, zeros_hbm, out_hbm,
             idx_v, rows_v, slots_v, outbuf_v, sem):
        c = lax.axis_index("c")
        s = lax.axis_index("s")
        wid = c * _NTILES + s
        pltpu.sync_copy(zeros_hbm, outbuf_v)
        iota = lax.iota(jnp.int32, 16)

        def chunk_body(ci, carry):
            pltpu.sync_copy(srcr.at[wid, ci], idx_v)
            pltpu.sync_copy(slotr.at[wid, ci], slots_v)
            descs = [pltpu.async_copy(m_hbm.at[idx_v.at[j]], rows_v.at[j], sem)
                     for j in range(_NSUB)]
            for d in descs:
                d.wait()

            def edge_body(eg, carry2):
                e0 = eg * 16
                gathered = []
                for u in range(16):
                    e = e0 + u
                    sub = e // 128
                    r = e - sub * 128
                    ev = jnp.full((16,), e, jnp.int32)
                    enc = plsc.load_gather(slots_v, [ev])
                    q32 = (enc >> 12) * 32
                    slot = enc & 0xFFF
                    sv = jnp.full((16,), sub, jnp.int32)
                    rv = jnp.full((16,), r, jnp.int32)
                    h0 = plsc.load_gather(rows_v, [sv, rv, q32 + iota])
                    h1 = plsc.load_gather(rows_v, [sv, rv, q32 + iota + 16])
                    gathered.append((slot, h0, h1))
                for slot, h0, h1 in gathered:
                    orow = slot >> 2
                    ocol = (slot & 3) * 32 + iota
                    plsc.addupdate_scatter(outbuf_v, [orow, ocol], h0)
                    plsc.addupdate_scatter(outbuf_v, [orow, ocol + 16], h1)
                return carry2

            lax.fori_loop(0, _CH // 16, edge_body, 0)
            return carry

        lax.fori_loop(0, _NCH, chunk_body, 0)
        pltpu.sync_copy(outbuf_v, out_hbm.at[wid])

    return spmm


def _get_spmm():
    if "spmm" not in _KCACHE:
        _KCACHE["spmm"] = _make_spmm()
    return _KCACHE["spmm"]


def _leaky(t):
    return jnp.where(t >= 0, t, 0.2 * t)


def _mm_body(h_ref, w_ref, o_ref):
    o_ref[0] = jnp.dot(h_ref[0], w_ref[0], preferred_element_type=jnp.float32)


_mm = pl.pallas_call(
    _mm_body,
    grid=(2,),
    in_specs=[pl.BlockSpec((1, _N, _C), lambda i: (i, 0, 0)),
              pl.BlockSpec((1, _C, _C), lambda i: (i, 0, 0))],
    out_specs=pl.BlockSpec((1, _N, _C), lambda i: (i, 0, 0)),
    out_shape=jax.ShapeDtypeStruct((2, _N, _C), jnp.float32),
)


def _gru_body(h_ref, agg_ref, wih_ref, whh_ref, bih_ref, bhh_ref, o_ref):
    h = h_ref[0]
    agg = agg_ref[0]
    gi = lax.dot_general(agg, wih_ref[0], (((1,), (1,)), ((), ())),
                         preferred_element_type=jnp.float32) + bih_ref[0]
    gh = lax.dot_general(h, whh_ref[0], (((1,), (1,)), ((), ())),
                         preferred_element_type=jnp.float32) + bhh_ref[0]
    r = jax.nn.sigmoid(gi[:, :_C] + gh[:, :_C])
    z = jax.nn.sigmoid(gi[:, _C:2 * _C] + gh[:, _C:2 * _C])
    cc = jnp.tanh(gi[:, 2 * _C:] + r * gh[:, 2 * _C:])
    o_ref[0] = (1.0 - z) * cc + z * h


_gru = pl.pallas_call(
    _gru_body,
    grid=(2,),
    in_specs=[pl.BlockSpec((1, _N, _C), lambda i: (i, 0, 0)),
              pl.BlockSpec((1, _N, _C), lambda i: (i, 0, 0)),
              pl.BlockSpec((1, 3 * _C, _C), lambda i: (i, 0, 0)),
              pl.BlockSpec((1, 3 * _C, _C), lambda i: (i, 0, 0)),
              pl.BlockSpec((1, 1, 3 * _C), lambda i: (i, 0, 0)),
              pl.BlockSpec((1, 1, 3 * _C), lambda i: (i, 0, 0))],
    out_specs=pl.BlockSpec((1, _N, _C), lambda i: (i, 0, 0)),
    out_shape=jax.ShapeDtypeStruct((2, _N, _C), jnp.float32),
)


def _leaky_body(h_ref, o_ref):
    o_ref[...] = _leaky(h_ref[...])


_leaky_tc = pl.pallas_call(
    _leaky_body,
    out_shape=jax.ShapeDtypeStruct((2, _N, _C), jnp.float32),
)


def _pool_body(oh_ref, h_ref, o_ref):
    o_ref[0] = jnp.dot(oh_ref[...], h_ref[0],
                       precision=lax.Precision.HIGHEST,
                       preferred_element_type=jnp.float32)


_pool = pl.pallas_call(
    _pool_body,
    grid=(2,),
    in_specs=[pl.BlockSpec((128, _N), lambda i: (0, 0)),
              pl.BlockSpec((1, _N, _C), lambda i: (i, 0, 0))],
    out_specs=pl.BlockSpec((1, 128, _C), lambda i: (i, 0, 0)),
    out_shape=jax.ShapeDtypeStruct((2, 128, _C), jnp.float32),
)


def _heads_body(hf_ref, pool_ref, inv_ref,
                tf1w, tf1b, tf2w, tf2b,
                b1w, b1b, b2w, b2b,
                v1w, v1b, v2w, v2b,
                nm1w, nm1b, nm2w, nm2b,
                nf1w, nf1b, nf2w, nf2b,
                t_ref, nm_ref, nf_ref, b_ref, v_ref):
    def mlp2(xx, w1, b1, w2, b2):
        y = _leaky(lax.dot_general(xx, w1[...], (((1,), (1,)), ((), ())),
                                   preferred_element_type=jnp.float32) + b1[...])
        return _leaky(lax.dot_general(y, w2[...], (((1,), (1,)), ((), ())),
                                      preferred_element_type=jnp.float32) + b2[...])

    pool_p = pool_ref[0] * inv_ref[...]
    pool_v = pool_ref[1] * inv_ref[...]
    t_ref[...] = mlp2(pool_p, tf1w, tf1b, tf2w, tf2b)
    b_ref[...] = mlp2(pool_p, b1w, b1b, b2w, b2b)
    v_ref[...] = mlp2(pool_v, v1w, v1b, v2w, v2b)
    nm_ref[...] = mlp2(hf_ref[0], nm1w, nm1b, nm2w, nm2b)
    nf_ref[...] = mlp2(hf_ref[0], nf1w, nf1b, nf2w, nf2b)


_heads = pl.pallas_call(
    _heads_body,
    out_shape=[
        jax.ShapeDtypeStruct((128, 128), jnp.float32),
        jax.ShapeDtypeStruct((_N, 128), jnp.float32),
        jax.ShapeDtypeStruct((_N, 128), jnp.float32),
        jax.ShapeDtypeStruct((128, 128), jnp.float32),
        jax.ShapeDtypeStruct((128, 128), jnp.float32),
    ],
)


def _pad_ggc(p):
    cc = p["w_hh"].shape[1]
    ll = p["W"].shape[0]
    wp = jnp.zeros((ll, _C, _C), jnp.float32).at[:, :cc, :cc].set(p["W"])
    wih = jnp.zeros((3 * _C, _C), jnp.float32)
    whh = jnp.zeros((3 * _C, _C), jnp.float32)
    bih = jnp.zeros((3 * _C,), jnp.float32)
    bhh = jnp.zeros((3 * _C,), jnp.float32)
    for g in range(3):
        wih = wih.at[g * _C:g * _C + cc, :cc].set(p["w_ih"][g * cc:(g + 1) * cc])
        whh = whh.at[g * _C:g * _C + cc, :cc].set(p["w_hh"][g * cc:(g + 1) * cc])
        bih = bih.at[g * _C:g * _C + cc].set(p["b_ih"][g * cc:(g + 1) * cc])
        bhh = bhh.at[g * _C:g * _C + cc].set(p["b_hh"][g * cc:(g + 1) * cc])
    return wp, wih, whh, bih, bhh


def _stack_stage(pa, pb):
    a = _pad_ggc(pa)
    b = _pad_ggc(pb)
    return (jnp.stack([a[0], b[0]], axis=1),
            jnp.stack([a[1], b[1]]),
            jnp.stack([a[2], b[2]]),
            jnp.stack([a[3], b[3]])[:, None, :],
            jnp.stack([a[4], b[4]])[:, None, :])


def _edge_setup(src, dst):
    perm = jnp.argsort(dst, stable=True)
    sd = dst[perm]
    sp = src[perm]

    nxt = jnp.concatenate([sd[1:], jnp.full((1,), -1, jnp.int32)])
    rend = jnp.zeros((_E,), jnp.bool_)
    for (a, b) in _RANGES:
        rend = rend.at[b - 1].set(True)
    flag = jnp.logical_or(sd != nxt, rend).astype(jnp.int32)
    g = jnp.cumsum(flag)
    pslot_full = jnp.zeros((_E,), jnp.int32)
    off1 = jnp.zeros((32,), jnp.int32)

    srcp = jnp.zeros((32, _RANGE_PAD), jnp.int32)
    enc0 = jnp.full((32, _RANGE_PAD), _PSLOT, jnp.int32)
    enc1 = jnp.full((32, _RANGE_PAD), _PSLOT, jnp.int32)
    for r, (a, b) in enumerate(_RANGES):
        n = b - a
        base = g[a - 1] if a > 0 else jnp.int32(0)
        nflush = g[b - 1] - base
        ps = jnp.minimum(g[a:b] - flag[a:b] - base, _S - 3)
        pslot_full = pslot_full.at[a:b].set(ps)
        off1 = off1.at[r].set(nflush)
        srcp = srcp.at[r, :n].set(sp[a:b])
        q0 = jnp.remainder(sp[a:b], 4)
        enc0 = enc0.at[r, :n].set(ps | (q0 << 12))
        ps1 = jnp.minimum(ps + nflush, _S - 3)
        q1 = jnp.remainder(sp[a:b] + _N, 4)
        enc1 = enc1.at[r, :n].set(ps1 | (q1 << 12))

    psrc0 = srcp >> 2
    psrc1 = (srcp + _N) >> 2
    srcr = jnp.concatenate([psrc0, psrc1], axis=1).reshape(
        32, _NCH, _NSUB, 128)
    slotr = jnp.concatenate([enc0, enc1], axis=1).reshape(32, _NCH, _CH)

    seg_start = jnp.searchsorted(sd, jnp.arange(_N, dtype=jnp.int32)
                                 ).astype(jnp.int32)
    seg_end = jnp.searchsorted(sd, jnp.arange(_N, dtype=jnp.int32),
                               side="right").astype(jnp.int32)
    has = seg_end > seg_start
    re_pos = jnp.asarray([b - 1 for (a, b) in _RANGES], jnp.int32)
    rs_pos = jnp.asarray([a for (a, b) in _RANGES], jnp.int32)
    j = jnp.searchsorted(re_pos, seg_start).astype(jnp.int32)
    cand = re_pos[jnp.minimum(j, 31)]
    e2 = seg_end - 1
    e1 = jnp.where(cand < e2, cand, e2)
    e1c = jnp.maximum(e1, 0)
    e2c = jnp.maximum(e2, 0)
    t1 = (jnp.searchsorted(rs_pos, e1c, side="right") - 1).astype(jnp.int32)
    t2 = (jnp.searchsorted(rs_pos, e2c, side="right") - 1).astype(jnp.int32)
    s1 = pslot_full[e1c]
    s2 = pslot_full[e2c]
    zs = jnp.int32(_ZSLOT)
    maps = []
    for tw in range(2):
        a1 = s1 + tw * off1[t1]
        a2 = s2 + tw * off1[t2]
        m1 = jnp.where(has, t1 * _S + jnp.minimum(a1, _S - 3), zs)
        m2 = jnp.where(has & (e1 != e2),
                       t2 * _S + jnp.minimum(a2, _S - 3), zs)
        maps.append((m1, m2))
    return srcr, slotr, maps


def kernel(x, edge_index, batch, params):
    src = edge_index[0].astype(jnp.int32)
    dst = edge_index[1].astype(jnp.int32)
    batch = batch.astype(jnp.int32)

    srcr, slotr, maps = _edge_setup(src, dst)
    map1 = jnp.stack([maps[0][0], maps[1][0]])
    map2 = jnp.stack([maps[0][1], maps[1][1]])
    zeros_buf = jnp.zeros((_S4, 128), jnp.float32)

    ends = jnp.searchsorted(batch, jnp.arange(_G, dtype=jnp.int32),
                            side="right").astype(jnp.int32)
    starts = jnp.concatenate([jnp.zeros((1,), jnp.int32), ends[:-1]])
    counts = ends - starts
    pos_from_end = jnp.take(ends, batch) - jnp.arange(_N, dtype=jnp.int32)
    keep_idx = jnp.nonzero(pos_from_end > 10, size=_N - 10 * _G)[0].astype(jnp.int32)
    inv_counts = 1.0 / jnp.maximum(counts, 1).astype(jnp.float32)
    inv_pad = jnp.ones((128, 1), jnp.float32).at[:_G, 0].set(inv_counts)
    onehot = (batch[None, :] == jnp.arange(128, dtype=jnp.int32)[:, None]
              ).astype(jnp.float32)

    stages = [
        _stack_stage(params["p_conv1"], params["v_conv1"]),
        _stack_stage(params["p_conv2"], params["v_conv2"]),
        _stack_stage(params["p_conv3"], params["v_conv3"]),
    ]

    h = jnp.pad(x, ((0, 0), (0, _C - x.shape[1])))
    h = jnp.broadcast_to(h[None], (2, _N, _C)).astype(jnp.float32)

    for wst, wih, whh, bih, bhh in stages:
        def body(hc, wl, wih=wih, whh=whh, bih=bih, bhh=bhh):
            m = _mm(hc, wl)
            part = _get_spmm()(m.reshape(_TROWS, 128), srcr, slotr, zeros_buf)
            pf = part.reshape(32, _S4, 4, 32).reshape(32 * _S, 32)
            agg = pf[map1] + pf[map2]
            hn = _gru(hc, agg, wih, whh, bih, bhh)
            return hn, None
        h, _ = lax.scan(body, h, wst)
        h = _leaky_tc(h)

    pool = _pool(onehot, h)

    pp = params
    ws = []
    for nm1, nm2 in (("t_fcn1", "t_fcn2"), ("b_fcn1", "b_fcn2"),
                     ("v_fcn1", "v_fcn2"), ("nmol_fcn1", "nmol_fcn2"),
                     ("nfull_fcn1", "nfull_fcn2")):
        w1p = jnp.zeros((128, 32), jnp.float32).at[:16, :].set(pp[nm1]["W"])
        b1p = jnp.zeros((1, 128), jnp.float32).at[0, :16].set(pp[nm1]["b"])
        wo = pp[nm2]["W"].shape[0]
        w2p = jnp.zeros((128, 128), jnp.float32).at[:wo, :16].set(pp[nm2]["W"])
        b2p = jnp.zeros((1, 128), jnp.float32).at[0, :wo].set(pp[nm2]["b"])
        ws += [w1p, b1p, w2p, b2p]

    t, nm_all, nf, b, v = _heads(h, pool, inv_pad, *ws)
    nm = jnp.take(nm_all[:, :1], keep_idx, axis=0)
    return (t[:_G, :2], nm, nf[:, :1], b[:_G, :3], v[:_G, :1])
